# baseline (speedup 1.0000x reference)
import jax
import jax.numpy as jnp
from jax import lax
from jax.experimental import pallas as pl
from jax.experimental.pallas import tpu as pltpu
from jax.experimental.pallas import tpu_sc as plsc

_B, _C, _K = 4096, 10000, 3
_BB = 128
_NW = 32
_RPW = _B // _NW


def _sc_gather_body(x_hbm, t0_hbm, t1_hbm, t2_hbm,
                    g0_hbm, g1_hbm, g2_hbm, m1_hbm, m2_hbm,
                    tv0, tv1, tv2, idx0, idx1, idx2, g0, g1, g2,
                    m1v, m2v, sem):
    info = plsc.get_sparse_core_info()
    nc = info.num_cores
    wid = lax.axis_index("s") * nc + lax.axis_index("c")
    base = wid * _RPW

    pltpu.sync_copy(t0_hbm.at[pl.ds(base, _RPW)], tv0)
    pltpu.sync_copy(t1_hbm.at[pl.ds(base, _RPW)], tv1)
    pltpu.sync_copy(t2_hbm.at[pl.ds(base, _RPW)], tv2)

    iota = lax.broadcasted_iota(jnp.int32, (16,), 0)
    one = jnp.ones((16,), jnp.float32)
    zero = jnp.zeros((16,), jnp.float32)
    for c in range(_RPW // 16):
        sl = pl.ds(c * 16, 16)
        t0v = tv0[sl]
        t1v = tv1[sl]
        t2v = tv2[sl]
        rows = (base + c * 16) + iota
        idx0[sl] = rows * _C + t0v
        idx1[sl] = rows * _C + t1v
        idx2[sl] = rows * _C + t2v
        m1v[sl] = jnp.where(t1v != t0v, one, zero)
        m2v[sl] = jnp.where((t2v != t0v) & (t2v != t1v), one, zero)

    pltpu.async_copy(x_hbm.at[idx0], g0, sem).wait()
    pltpu.async_copy(x_hbm.at[idx1], g1, sem).wait()
    pltpu.async_copy(x_hbm.at[idx2], g2, sem).wait()

    pltpu.sync_copy(g0, g0_hbm.at[pl.ds(base, _RPW)])
    pltpu.sync_copy(g1, g1_hbm.at[pl.ds(base, _RPW)])
    pltpu.sync_copy(g2, g2_hbm.at[pl.ds(base, _RPW)])
    pltpu.sync_copy(m1v, m1_hbm.at[pl.ds(base, _RPW)])
    pltpu.sync_copy(m2v, m2_hbm.at[pl.ds(base, _RPW)])


def _sc_gather(x_flat, t0, t1, t2):
    mesh = plsc.VectorSubcoreMesh(core_axis_name="c", subcore_axis_name="s")
    vec = jax.ShapeDtypeStruct((_B,), jnp.float32)
    return pl.kernel(
        _sc_gather_body,
        mesh=mesh,
        out_type=[vec, vec, vec, vec, vec],
        scratch_types=[
            pltpu.VMEM((_RPW,), jnp.int32),
            pltpu.VMEM((_RPW,), jnp.int32),
            pltpu.VMEM((_RPW,), jnp.int32),
            pltpu.VMEM((_RPW,), jnp.int32),
            pltpu.VMEM((_RPW,), jnp.int32),
            pltpu.VMEM((_RPW,), jnp.int32),
            pltpu.VMEM((_RPW,), jnp.float32),
            pltpu.VMEM((_RPW,), jnp.float32),
            pltpu.VMEM((_RPW,), jnp.float32),
            pltpu.VMEM((_RPW,), jnp.float32),
            pltpu.VMEM((_RPW,), jnp.float32),
            pltpu.SemaphoreType.DMA,
        ],
    )(x_flat, t0, t1, t2)


def _dense_body(x_ref, g0_ref, g1_ref, g2_ref, m1_ref, m2_ref, out_ref):
    i = pl.program_id(0)
    v = x_ref[...]
    x = jax.nn.log_sigmoid(v)
    recip = jnp.sum(1.0 / x, axis=1, keepdims=True)
    ls0 = jax.nn.log_sigmoid(g0_ref[...])
    ls1 = jax.nn.log_sigmoid(g1_ref[...])
    ls2 = jax.nn.log_sigmoid(g2_ref[...])
    ts = ls0 + m1_ref[...] * ls1 + m2_ref[...] * ls2
    ts = jnp.maximum(ts, 1e-6)
    partial = jax.lax.dot_general(
        ts, recip, (((1,), (0,)), ((), ())),
        precision=jax.lax.Precision.HIGHEST,
        preferred_element_type=jnp.float32)

    @pl.when(i == 0)
    def _init():
        out_ref[...] = jnp.zeros((1, 1), jnp.float32)

    out_ref[...] += partial


def kernel(inputs, targets):
    x2d = inputs[..., 0]
    g0, g1, g2, m1, m2 = _sc_gather(
        x2d.reshape(_B * _C),
        targets[:, 0], targets[:, 1], targets[:, 2],
    )
    grid = _B // _BB
    row_spec = pl.BlockSpec((1, _BB), lambda i: (0, i))
    acc = pl.pallas_call(
        _dense_body,
        grid=(grid,),
        in_specs=[
            pl.BlockSpec((_BB, _C), lambda i: (i, 0)),
            row_spec, row_spec, row_spec, row_spec, row_spec,
        ],
        out_specs=pl.BlockSpec((1, 1), lambda i: (0, 0)),
        out_shape=jax.ShapeDtypeStruct((1, 1), jnp.float32),
        compiler_params=pltpu.CompilerParams(
            dimension_semantics=("arbitrary",),
        ),
    )(x2d, g0.reshape(1, _B), g1.reshape(1, _B), g2.reshape(1, _B),
      m1.reshape(1, _B), m2.reshape(1, _B))
    return -acc[0, 0] / _B


# speedup vs baseline: 1.0211x; 1.0211x over previous
import jax
import jax.numpy as jnp
from jax import lax
from jax.experimental import pallas as pl
from jax.experimental.pallas import tpu as pltpu
from jax.experimental.pallas import tpu_sc as plsc

_B, _C, _K = 4096, 10000, 3
_BB = 128
_NW = 32
_RPW = _B // _NW


def _sc_gather_body(x_hbm, t0_hbm, t1_hbm, t2_hbm,
                    g0_hbm, g1_hbm, g2_hbm, m1_hbm, m2_hbm,
                    tv0, tv1, tv2, idx0, idx1, idx2, g0, g1, g2,
                    m1v, m2v, sem):
    info = plsc.get_sparse_core_info()
    nc = info.num_cores
    wid = lax.axis_index("s") * nc + lax.axis_index("c")
    base = wid * _RPW

    pltpu.sync_copy(t0_hbm.at[pl.ds(base, _RPW)], tv0)
    pltpu.sync_copy(t1_hbm.at[pl.ds(base, _RPW)], tv1)
    pltpu.sync_copy(t2_hbm.at[pl.ds(base, _RPW)], tv2)

    iota = lax.broadcasted_iota(jnp.int32, (16,), 0)
    one = jnp.ones((16,), jnp.float32)
    zero = jnp.zeros((16,), jnp.float32)
    for c in range(_RPW // 16):
        sl = pl.ds(c * 16, 16)
        t0v = tv0[sl]
        t1v = tv1[sl]
        t2v = tv2[sl]
        rows = (base + c * 16) + iota
        idx0[sl] = rows * _C + t0v
        idx1[sl] = rows * _C + t1v
        idx2[sl] = rows * _C + t2v
        m1v[sl] = jnp.where(t1v != t0v, one, zero)
        m2v[sl] = jnp.where((t2v != t0v) & (t2v != t1v), one, zero)

    pltpu.async_copy(x_hbm.at[idx0], g0, sem).wait()
    pltpu.async_copy(x_hbm.at[idx1], g1, sem).wait()
    pltpu.async_copy(x_hbm.at[idx2], g2, sem).wait()

    pltpu.sync_copy(g0, g0_hbm.at[pl.ds(base, _RPW)])
    pltpu.sync_copy(g1, g1_hbm.at[pl.ds(base, _RPW)])
    pltpu.sync_copy(g2, g2_hbm.at[pl.ds(base, _RPW)])
    pltpu.sync_copy(m1v, m1_hbm.at[pl.ds(base, _RPW)])
    pltpu.sync_copy(m2v, m2_hbm.at[pl.ds(base, _RPW)])


def _sc_gather(x_flat, t0, t1, t2):
    mesh = plsc.VectorSubcoreMesh(core_axis_name="c", subcore_axis_name="s")
    vec = jax.ShapeDtypeStruct((_B,), jnp.float32)
    return pl.kernel(
        _sc_gather_body,
        mesh=mesh,
        out_type=[vec, vec, vec, vec, vec],
        scratch_types=[
            pltpu.VMEM((_RPW,), jnp.int32),
            pltpu.VMEM((_RPW,), jnp.int32),
            pltpu.VMEM((_RPW,), jnp.int32),
            pltpu.VMEM((_RPW,), jnp.int32),
            pltpu.VMEM((_RPW,), jnp.int32),
            pltpu.VMEM((_RPW,), jnp.int32),
            pltpu.VMEM((_RPW,), jnp.float32),
            pltpu.VMEM((_RPW,), jnp.float32),
            pltpu.VMEM((_RPW,), jnp.float32),
            pltpu.VMEM((_RPW,), jnp.float32),
            pltpu.VMEM((_RPW,), jnp.float32),
            pltpu.SemaphoreType.DMA,
        ],
    )(x_flat, t0, t1, t2)


def _dense_body(x_ref, out_ref):
    v = x_ref[...]
    x = jax.nn.log_sigmoid(v)
    out_ref[...] = jnp.sum(1.0 / x, axis=1, keepdims=True)


def _combine_body(rs_ref, g0_ref, g1_ref, g2_ref, m1_ref, m2_ref, out_ref):
    ls0 = jax.nn.log_sigmoid(g0_ref[...])
    ls1 = jax.nn.log_sigmoid(g1_ref[...])
    ls2 = jax.nn.log_sigmoid(g2_ref[...])
    ts = ls0 + m1_ref[...] * ls1 + m2_ref[...] * ls2
    ts = jnp.maximum(ts, 1e-6)
    dot = jax.lax.dot_general(
        ts, rs_ref[...], (((1,), (0,)), ((), ())),
        precision=jax.lax.Precision.HIGHEST,
        preferred_element_type=jnp.float32)
    out_ref[...] = -dot / _B


def kernel(inputs, targets):
    x2d = inputs[..., 0]
    g0, g1, g2, m1, m2 = _sc_gather(
        x2d.reshape(_B * _C),
        targets[:, 0], targets[:, 1], targets[:, 2],
    )
    grid = _B // _BB
    rowsum = pl.pallas_call(
        _dense_body,
        grid=(grid,),
        in_specs=[pl.BlockSpec((_BB, _C), lambda i: (i, 0))],
        out_specs=pl.BlockSpec((_BB, 1), lambda i: (i, 0)),
        out_shape=jax.ShapeDtypeStruct((_B, 1), jnp.float32),
        compiler_params=pltpu.CompilerParams(
            dimension_semantics=("arbitrary",),
        ),
    )(x2d)
    row_spec = pl.BlockSpec((1, _B), lambda: (0, 0))
    acc = pl.pallas_call(
        _combine_body,
        in_specs=[
            pl.BlockSpec((_B, 1), lambda: (0, 0)),
            row_spec, row_spec, row_spec, row_spec, row_spec,
        ],
        out_specs=pl.BlockSpec((1, 1), lambda: (0, 0)),
        out_shape=jax.ShapeDtypeStruct((1, 1), jnp.float32),
    )(rowsum, g0.reshape(1, _B), g1.reshape(1, _B), g2.reshape(1, _B),
      m1.reshape(1, _B), m2.reshape(1, _B))
    return acc[0, 0]


# speedup vs baseline: 1.3530x; 1.3250x over previous
import jax
import jax.numpy as jnp
from jax.experimental import pallas as pl
from jax.experimental.pallas import tpu as pltpu

_B, _C, _K = 4096, 10000, 3
_BB = 128


def _loss_body(x_ref, t_ref, out_ref):
    i = pl.program_id(0)
    v = x_ref[...]
    x = jax.nn.log_sigmoid(v)
    recip = jnp.sum(1.0 / x, axis=1)
    tb = t_ref[...]
    cols = jax.lax.broadcasted_iota(jnp.int32, (_BB, _C), 1)
    mask = (cols == tb[:, 0:1]) | (cols == tb[:, 1:2]) | (cols == tb[:, 2:3])
    ts = jnp.maximum(jnp.sum(jnp.where(mask, x, 0.0), axis=1), 1e-6)
    partial = jnp.sum(ts * recip).reshape(1, 1)

    @pl.when(i == 0)
    def _init():
        out_ref[...] = jnp.zeros((1, 1), jnp.float32)

    out_ref[...] += partial


def kernel(inputs, targets):
    x2d = inputs[..., 0]
    grid = _B // _BB
    acc = pl.pallas_call(
        _loss_body,
        grid=(grid,),
        in_specs=[
            pl.BlockSpec((_BB, _C), lambda i: (i, 0)),
            pl.BlockSpec((_BB, _K), lambda i: (i, 0)),
        ],
        out_specs=pl.BlockSpec((1, 1), lambda i: (0, 0)),
        out_shape=jax.ShapeDtypeStruct((1, 1), jnp.float32),
        compiler_params=pltpu.CompilerParams(
            dimension_semantics=("arbitrary",),
        ),
    )(x2d, targets)
    return -acc[0, 0] / _B


# speedup vs baseline: 3.1220x; 2.3075x over previous
import jax
import jax.numpy as jnp
from jax import lax
from jax.experimental import pallas as pl
from jax.experimental.pallas import tpu as pltpu
from jax.experimental.pallas import tpu_sc as plsc

_B, _C, _K = 4096, 10000, 3
_BH = _B // 128
_CB = 625
_NW = 32
_RPW = _B // _NW


def _sc_gather_body(x_hbm, t0_hbm, t1_hbm, t2_hbm,
                    g0_hbm, g1_hbm, g2_hbm, m1_hbm, m2_hbm,
                    tv0, tv1, tv2, idx0, idx1, idx2, g0, g1, g2,
                    m1v, m2v, sem):
    info = plsc.get_sparse_core_info()
    nc = info.num_cores
    wid = lax.axis_index("s") * nc + lax.axis_index("c")
    base = wid * _RPW

    pltpu.sync_copy(t0_hbm.at[pl.ds(base, _RPW)], tv0)
    pltpu.sync_copy(t1_hbm.at[pl.ds(base, _RPW)], tv1)
    pltpu.sync_copy(t2_hbm.at[pl.ds(base, _RPW)], tv2)

    iota = lax.broadcasted_iota(jnp.int32, (16,), 0)
    one = jnp.ones((16,), jnp.float32)
    zero = jnp.zeros((16,), jnp.float32)
    for c in range(_RPW // 16):
        sl = pl.ds(c * 16, 16)
        t0v = tv0[sl]
        t1v = tv1[sl]
        t2v = tv2[sl]
        rows = (base + c * 16) + iota
        idx0[sl] = t0v * _B + rows
        idx1[sl] = t1v * _B + rows
        idx2[sl] = t2v * _B + rows
        m1v[sl] = jnp.where(t1v != t0v, one, zero)
        m2v[sl] = jnp.where((t2v != t0v) & (t2v != t1v), one, zero)

    pltpu.async_copy(x_hbm.at[idx0], g0, sem).wait()
    pltpu.async_copy(x_hbm.at[idx1], g1, sem).wait()
    pltpu.async_copy(x_hbm.at[idx2], g2, sem).wait()

    pltpu.sync_copy(g0, g0_hbm.at[pl.ds(base, _RPW)])
    pltpu.sync_copy(g1, g1_hbm.at[pl.ds(base, _RPW)])
    pltpu.sync_copy(g2, g2_hbm.at[pl.ds(base, _RPW)])
    pltpu.sync_copy(m1v, m1_hbm.at[pl.ds(base, _RPW)])
    pltpu.sync_copy(m2v, m2_hbm.at[pl.ds(base, _RPW)])


def _sc_gather(xt_flat, t0, t1, t2):
    mesh = plsc.VectorSubcoreMesh(core_axis_name="c", subcore_axis_name="s")
    vec = jax.ShapeDtypeStruct((_B,), jnp.float32)
    return pl.kernel(
        _sc_gather_body,
        mesh=mesh,
        out_type=[vec, vec, vec, vec, vec],
        scratch_types=[
            pltpu.VMEM((_RPW,), jnp.int32),
            pltpu.VMEM((_RPW,), jnp.int32),
            pltpu.VMEM((_RPW,), jnp.int32),
            pltpu.VMEM((_RPW,), jnp.int32),
            pltpu.VMEM((_RPW,), jnp.int32),
            pltpu.VMEM((_RPW,), jnp.int32),
            pltpu.VMEM((_RPW,), jnp.float32),
            pltpu.VMEM((_RPW,), jnp.float32),
            pltpu.VMEM((_RPW,), jnp.float32),
            pltpu.VMEM((_RPW,), jnp.float32),
            pltpu.VMEM((_RPW,), jnp.float32),
            pltpu.SemaphoreType.DMA,
        ],
    )(xt_flat, t0, t1, t2)


def _dense_body(x_ref, g0_ref, g1_ref, g2_ref, m1_ref, m2_ref,
                out_ref, acc_ref):
    i = pl.program_id(0)
    psum = jnp.zeros((_BH, 128), jnp.float32)
    for k in range(_CB // _SUB):
        blk = x_ref[pl.ds(k * _SUB, _SUB)]
        psum = psum + jnp.sum(1.0 / jax.nn.log_sigmoid(blk), axis=0)

    @pl.when(i == 0)
    def _init():
        acc_ref[...] = jnp.zeros((_BH, 128), jnp.float32)

    acc_ref[...] += psum

    @pl.when(i == pl.num_programs(0) - 1)
    def _fin():
        ls0 = jax.nn.log_sigmoid(g0_ref[...])
        ls1 = jax.nn.log_sigmoid(g1_ref[...])
        ls2 = jax.nn.log_sigmoid(g2_ref[...])
        ts = ls0 + m1_ref[...] * ls1 + m2_ref[...] * ls2
        ts = jnp.maximum(ts, 1e-6)
        out_ref[...] = (-jnp.sum(ts * acc_ref[...]) / _B).reshape(1, 1)


def kernel(inputs, targets):
    xt = jnp.transpose(inputs[..., 0], (1, 0))
    g0, g1, g2, m1, m2 = _sc_gather(
        xt.reshape(_C * _B),
        targets[:, 0], targets[:, 1], targets[:, 2],
    )
    lane_spec = pl.BlockSpec((_BH, 128), lambda i: (0, 0))
    acc = pl.pallas_call(
        _dense_body,
        grid=(_C // _CB,),
        in_specs=[
            pl.BlockSpec((_CB, _BH, 128), lambda i: (i, 0, 0)),
            lane_spec, lane_spec, lane_spec, lane_spec, lane_spec,
        ],
        out_specs=pl.BlockSpec((1, 1), lambda i: (0, 0)),
        out_shape=jax.ShapeDtypeStruct((1, 1), jnp.float32),
        scratch_shapes=[pltpu.VMEM((_BH, 128), jnp.float32)],
        compiler_params=pltpu.CompilerParams(
            dimension_semantics=("arbitrary",),
        ),
    )(xt.reshape(_C, _BH, 128),
      g0.reshape(_BH, 128), g1.reshape(_BH, 128), g2.reshape(_BH, 128),
      m1.reshape(_BH, 128), m2.reshape(_BH, 128))
    return acc[0, 0]
